# baseline (speedup 1.0000x reference)
import jax
import jax.numpy as jnp
from jax import lax
from jax.experimental import pallas as pl
from jax.experimental.pallas import tpu as pltpu
from jax.experimental.pallas import tpu_sc as plsc

_D = 64
_BATCH = 1024
_SEQ = 1024
_B = _BATCH * _SEQ
_V = 1000000
_NC = 2
_NS = 16
_NW = _NC * _NS
_BPW = _B // _NW
_IW = 128
_NBLK = _V // _IW
_TAILV = _NBLK * _IW
_ROWS_PW = _BPW // _SEQ


def _fmt_body(tblt, tailp, ids, fmt, idxh, ibuf0, ibuf1, obuf0, obuf1,
              idsblk, idxblk, isem0, isem1, osem0, osem1):
    wid = lax.axis_index("s") * _NC + lax.axis_index("c")

    iota16 = lax.iota(jnp.int32, 16)

    def transpose_block(ibuf, obuf, nrows):
        def krow(k, carry):
            for p in range(2):
                col = jnp.broadcast_to(2 * k + p, (16,)).astype(jnp.int32)
                for dj in range(4):
                    vals = plsc.load_gather(ibuf, [iota16 + 16 * dj, col])
                    obuf[k, pl.ds(p * 64 + 16 * dj, 16)] = vals
            return carry

        lax.fori_loop(0, nrows, krow, 0)

    def sup_body(sup, carry):
        r0 = wid * _ROWS_PW + sup * 8
        pltpu.sync_copy(ids.at[pl.ds(r0, 8)], idsblk)

        def shift16(i, carry2):
            r = i // 64
            c = (i % 64) * 16
            idxblk[r, pl.ds(c, 16)] = lax.shift_right_logical(
                idsblk[r, pl.ds(c, 16)], 1)
            return carry2

        lax.fori_loop(0, 512, shift16, 0)
        pltpu.sync_copy(idxblk, idxh.at[pl.ds(r0, 8)])
        return carry

    lax.fori_loop(0, _ROWS_PW // 8, sup_body, 0)

    def fire_in(c, ibuf, sem):
        pltpu.async_copy(tblt.at[:, pl.ds(c * _IW, _IW)], ibuf, sem)

    def drain_in(ibuf, sem):
        pltpu.make_async_copy(tblt.at[:, pl.ds(0, _IW)], ibuf, sem).wait()

    def fire_out(c, obuf, sem):
        pltpu.async_copy(obuf, fmt.at[pl.ds(c * 64, 64)], sem)

    def drain_out(obuf, sem):
        pltpu.make_async_copy(obuf, fmt.at[pl.ds(0, 64)], sem).wait()

    nblk_w = _NBLK // _NW
    c0 = wid * nblk_w
    fire_in(c0, ibuf0, isem0)

    def pairstep(k, carry):
        c = c0 + 2 * k
        drain_in(ibuf0, isem0)
        fire_in(c + 1, ibuf1, isem1)
        transpose_block(ibuf0, obuf0, 64)
        fire_out(c, obuf0, osem0)
        drain_in(ibuf1, isem1)

        @pl.when(c + 2 < c0 + nblk_w)
        def _():
            fire_in(c + 2, ibuf0, isem0)

        transpose_block(ibuf1, obuf1, 64)
        fire_out(c + 1, obuf1, osem1)
        drain_out(obuf0, osem0)
        drain_out(obuf1, osem1)
        return carry

    lax.fori_loop(0, nblk_w // 2, pairstep, 0)

    @pl.when(wid < _NBLK - nblk_w * _NW)
    def _():
        c = nblk_w * _NW + wid
        pltpu.sync_copy(tblt.at[:, pl.ds(c * _IW, _IW)], ibuf0)
        transpose_block(ibuf0, obuf0, 64)
        pltpu.sync_copy(obuf0, fmt.at[pl.ds(c * 64, 64)])

    @pl.when(wid == _NW - 1)
    def _():
        pltpu.sync_copy(tailp, ibuf0)
        transpose_block(ibuf0, obuf0, 32)
        pltpu.sync_copy(obuf0.at[pl.ds(0, 32)], fmt.at[pl.ds(_TAILV // 2, 32)])


def _gather_body(ids, idxh, fmt, out3, idsblk, idxblk, pairs0, pairs1,
                 obuf0, obuf1, gsem0, gsem1, wsem0, wsem1):
    wid = lax.axis_index("s") * _NC + lax.axis_index("c")
    row0 = wid * _ROWS_PW

    iota16 = lax.iota(jnp.int32, 16)
    one16 = jnp.broadcast_to(jnp.int32(1), (16,))

    def fire_g(cc, pairs, sem):
        r = cc // 8
        c = (cc % 8) * _IW
        pltpu.async_copy(fmt.at[idxblk.at[r, pl.ds(c, _IW)]], pairs, sem)

    def drain_g(pairs, sem):
        pltpu.make_async_copy(fmt.at[pl.ds(0, _IW)], pairs, sem).wait()

    def select_t(cc, pairs, obuf):
        r = cc // 8
        c = (cc % 8) * _IW
        for g in range(8):
            rowidx = iota16 + 16 * g
            half = lax.shift_left(
                lax.bitwise_and(idsblk[r, pl.ds(c + 16 * g, 16)], 1), 6)

            def dloop(dd, col):
                for dj in range(8):
                    vals = plsc.load_gather(pairs, [rowidx, col])
                    obuf[dd * 8 + dj, pl.ds(16 * g, 16)] = vals
                    col = col + one16
                return col

            lax.fori_loop(0, 8, dloop, half)

    def fire_w(cc, sup, obuf, sem):
        row = row0 + sup * 8 + cc // 8
        pltpu.async_copy(obuf, out3.at[row, :, pl.ds((cc % 8) * _IW, _IW)], sem)

    def drain_w(obuf, sem):
        pltpu.make_async_copy(obuf, out3.at[0, :, pl.ds(0, _IW)], sem).wait()

    def sup_body(sup, carry):
        r0 = row0 + sup * 8
        pltpu.sync_copy(ids.at[pl.ds(r0, 8)], idsblk)
        pltpu.sync_copy(idxh.at[pl.ds(r0, 8)], idxblk)

        fire_g(0, pairs0, gsem0)

        def pairstep(k, carry2):
            cc = 2 * k
            drain_g(pairs0, gsem0)
            fire_g(cc + 1, pairs1, gsem1)
            select_t(cc, pairs0, obuf0)
            fire_w(cc, sup, obuf0, wsem0)
            drain_g(pairs1, gsem1)

            @pl.when(cc + 2 < 64)
            def _():
                fire_g(cc + 2, pairs0, gsem0)

            select_t(cc + 1, pairs1, obuf1)
            fire_w(cc + 1, sup, obuf1, wsem1)
            drain_w(obuf0, wsem0)
            drain_w(obuf1, wsem1)
            return carry2

        lax.fori_loop(0, 32, pairstep, 0)
        return carry

    lax.fori_loop(0, _ROWS_PW // 8, sup_body, 0)


@jax.jit
def kernel(input_ids, attention_mask, embedding_weight):
    tblt = embedding_weight.T
    tail = lax.slice(embedding_weight, (_TAILV, 0), (_V, _D))
    tailp = jnp.concatenate([tail.T, tail.T], axis=1)

    mesh = plsc.VectorSubcoreMesh(core_axis_name="c", subcore_axis_name="s")
    fmt, idxh = pl.kernel(
        _fmt_body,
        mesh=mesh,
        out_type=(
            jax.ShapeDtypeStruct((_V // 2, 128), jnp.float32),
            jax.ShapeDtypeStruct((_BATCH, _SEQ), jnp.int32),
        ),
        scratch_types=[
            pltpu.VMEM((_D, _IW), jnp.float32),
            pltpu.VMEM((_D, _IW), jnp.float32),
            pltpu.VMEM((_D, _IW), jnp.float32),
            pltpu.VMEM((_D, _IW), jnp.float32),
            pltpu.VMEM((8, _SEQ), jnp.int32),
            pltpu.VMEM((8, _SEQ), jnp.int32),
            pltpu.SemaphoreType.DMA,
            pltpu.SemaphoreType.DMA,
            pltpu.SemaphoreType.DMA,
            pltpu.SemaphoreType.DMA,
        ],
        compiler_params=pltpu.CompilerParams(use_tc_tiling_on_sc=True, needs_layout_passes=False),
    )(tblt, tailp, input_ids)

    out3 = pl.kernel(
        _gather_body,
        mesh=mesh,
        out_type=jax.ShapeDtypeStruct((_BATCH, _D, _SEQ), jnp.float32),
        scratch_types=[
            pltpu.VMEM((8, _SEQ), jnp.int32),
            pltpu.VMEM((8, _SEQ), jnp.int32),
            pltpu.VMEM((_IW, 128), jnp.float32),
            pltpu.VMEM((_IW, 128), jnp.float32),
            pltpu.VMEM((_D, _IW), jnp.float32),
            pltpu.VMEM((_D, _IW), jnp.float32),
            pltpu.SemaphoreType.DMA,
            pltpu.SemaphoreType.DMA,
            pltpu.SemaphoreType.DMA,
            pltpu.SemaphoreType.DMA,
        ],
        compiler_params=pltpu.CompilerParams(use_tc_tiling_on_sc=True, needs_layout_passes=False),
    )(input_ids, idxh, fmt)

    token_embeddings = out3.transpose(0, 2, 1)
    return (input_ids, token_embeddings, attention_mask)


# speedup vs baseline: 1.4907x; 1.4907x over previous
import jax
import jax.numpy as jnp
from jax import lax
from jax.experimental import pallas as pl
from jax.experimental.pallas import tpu as pltpu
from jax.experimental.pallas import tpu_sc as plsc

_D = 64
_BATCH = 1024
_SEQ = 1024
_B = _BATCH * _SEQ
_V = 1000000
_NC = 2
_NS = 16
_NW = _NC * _NS
_BPW = _B // _NW
_IW = 128
_NBLK = _V // _IW
_TAILV = _NBLK * _IW
_ROWS_PW = _BPW // _SEQ


def _fmt_body(tblt, tailp, ids, fmt, idxh, ibuf0, ibuf1, obuf0, obuf1,
              idsblk, idxblk, isem0, isem1, osem0, osem1):
    wid = lax.axis_index("s") * _NC + lax.axis_index("c")

    iota16 = lax.iota(jnp.int32, 16)

    def transpose_block(ibuf, obuf, nrows):
        rowi = [iota16 + 16 * dj for dj in range(4)]

        def krow4(k4, carry):
            k0 = k4 * 4
            vals = []
            for ku in range(4):
                for p in range(2):
                    col = jnp.broadcast_to(2 * (k0 + ku) + p, (16,)).astype(
                        jnp.int32)
                    for dj in range(4):
                        vals.append((k0 + ku, p * 64 + 16 * dj,
                                     plsc.load_gather(ibuf, [rowi[dj], col])))
            for k, off, v in vals:
                obuf[k, pl.ds(off, 16)] = v
            return carry

        lax.fori_loop(0, nrows // 4, krow4, 0)

    def sup_body(sup, carry):
        r0 = wid * _ROWS_PW + sup * 8
        pltpu.sync_copy(ids.at[pl.ds(r0, 8)], idsblk)

        def shift16(i, carry2):
            r = i // 64
            c = (i % 64) * 16
            idxblk[r, pl.ds(c, 16)] = lax.shift_right_logical(
                idsblk[r, pl.ds(c, 16)], 1)
            return carry2

        lax.fori_loop(0, 512, shift16, 0)
        pltpu.sync_copy(idxblk, idxh.at[pl.ds(r0, 8)])
        return carry

    lax.fori_loop(0, _ROWS_PW // 8, sup_body, 0)

    def fire_in(c, ibuf, sem):
        pltpu.async_copy(tblt.at[:, pl.ds(c * _IW, _IW)], ibuf, sem)

    def drain_in(ibuf, sem):
        pltpu.make_async_copy(tblt.at[:, pl.ds(0, _IW)], ibuf, sem).wait()

    def fire_out(c, obuf, sem):
        pltpu.async_copy(obuf, fmt.at[pl.ds(c * 64, 64)], sem)

    def drain_out(obuf, sem):
        pltpu.make_async_copy(obuf, fmt.at[pl.ds(0, 64)], sem).wait()

    nblk_w = _NBLK // _NW
    c0 = wid * nblk_w
    fire_in(c0, ibuf0, isem0)

    def pairstep(k, carry):
        c = c0 + 2 * k
        drain_in(ibuf0, isem0)
        fire_in(c + 1, ibuf1, isem1)
        transpose_block(ibuf0, obuf0, 64)
        fire_out(c, obuf0, osem0)
        drain_in(ibuf1, isem1)

        @pl.when(c + 2 < c0 + nblk_w)
        def _():
            fire_in(c + 2, ibuf0, isem0)

        transpose_block(ibuf1, obuf1, 64)
        fire_out(c + 1, obuf1, osem1)
        drain_out(obuf0, osem0)
        drain_out(obuf1, osem1)
        return carry

    lax.fori_loop(0, nblk_w // 2, pairstep, 0)

    @pl.when(wid < _NBLK - nblk_w * _NW)
    def _():
        c = nblk_w * _NW + wid
        pltpu.sync_copy(tblt.at[:, pl.ds(c * _IW, _IW)], ibuf0)
        transpose_block(ibuf0, obuf0, 64)
        pltpu.sync_copy(obuf0, fmt.at[pl.ds(c * 64, 64)])

    @pl.when(wid == _NW - 1)
    def _():
        pltpu.sync_copy(tailp, ibuf0)
        transpose_block(ibuf0, obuf0, 32)
        pltpu.sync_copy(obuf0.at[pl.ds(0, 32)], fmt.at[pl.ds(_TAILV // 2, 32)])


def _gather_body(ids, idxh, fmt, out3, idsblk, idxblk, pairs0, pairs1,
                 obuf0, obuf1, gsem0, gsem1, wsem0, wsem1):
    wid = lax.axis_index("s") * _NC + lax.axis_index("c")
    row0 = wid * _ROWS_PW

    iota16 = lax.iota(jnp.int32, 16)
    one16 = jnp.broadcast_to(jnp.int32(1), (16,))

    def fire_g(cc, pairs, sem):
        r = cc // 8
        c = (cc % 8) * _IW
        pltpu.async_copy(fmt.at[idxblk.at[r, pl.ds(c, _IW)]], pairs, sem)

    def drain_g(pairs, sem):
        pltpu.make_async_copy(fmt.at[pl.ds(0, _IW)], pairs, sem).wait()

    def select_t(cc, pairs, obuf):
        r = cc // 8
        c = (cc % 8) * _IW
        for g in range(8):
            rowidx = iota16 + 16 * g
            half = lax.shift_left(
                lax.bitwise_and(idsblk[r, pl.ds(c + 16 * g, 16)], 1), 6)

            def dloop(dd, col):
                vals = []
                for dj in range(32):
                    vals.append(plsc.load_gather(pairs, [rowidx, col + dj * one16]))
                for dj in range(32):
                    obuf[dd * 32 + dj, pl.ds(16 * g, 16)] = vals[dj]
                return col + 32 * one16

            lax.fori_loop(0, 2, dloop, half)

    def fire_w(cc, sup, obuf, sem):
        row = row0 + sup * 8 + cc // 8
        pltpu.async_copy(obuf, out3.at[row, :, pl.ds((cc % 8) * _IW, _IW)], sem)

    def drain_w(obuf, sem):
        pltpu.make_async_copy(obuf, out3.at[0, :, pl.ds(0, _IW)], sem).wait()

    def sup_body(sup, carry):
        r0 = row0 + sup * 8
        pltpu.sync_copy(ids.at[pl.ds(r0, 8)], idsblk)
        pltpu.sync_copy(idxh.at[pl.ds(r0, 8)], idxblk)

        fire_g(0, pairs0, gsem0)

        def pairstep(k, carry2):
            cc = 2 * k
            drain_g(pairs0, gsem0)
            fire_g(cc + 1, pairs1, gsem1)
            select_t(cc, pairs0, obuf0)
            fire_w(cc, sup, obuf0, wsem0)
            drain_g(pairs1, gsem1)

            @pl.when(cc + 2 < 64)
            def _():
                fire_g(cc + 2, pairs0, gsem0)

            select_t(cc + 1, pairs1, obuf1)
            fire_w(cc + 1, sup, obuf1, wsem1)
            drain_w(obuf0, wsem0)
            drain_w(obuf1, wsem1)
            return carry2

        lax.fori_loop(0, 32, pairstep, 0)
        return carry

    lax.fori_loop(0, _ROWS_PW // 8, sup_body, 0)


@jax.jit
def kernel(input_ids, attention_mask, embedding_weight):
    tblt = embedding_weight.T
    tail = lax.slice(embedding_weight, (_TAILV, 0), (_V, _D))
    tailp = jnp.concatenate([tail.T, tail.T], axis=1)

    mesh = plsc.VectorSubcoreMesh(core_axis_name="c", subcore_axis_name="s")
    fmt, idxh = pl.kernel(
        _fmt_body,
        mesh=mesh,
        out_type=(
            jax.ShapeDtypeStruct((_V // 2, 128), jnp.float32),
            jax.ShapeDtypeStruct((_BATCH, _SEQ), jnp.int32),
        ),
        scratch_types=[
            pltpu.VMEM((_D, _IW), jnp.float32),
            pltpu.VMEM((_D, _IW), jnp.float32),
            pltpu.VMEM((_D, _IW), jnp.float32),
            pltpu.VMEM((_D, _IW), jnp.float32),
            pltpu.VMEM((8, _SEQ), jnp.int32),
            pltpu.VMEM((8, _SEQ), jnp.int32),
            pltpu.SemaphoreType.DMA,
            pltpu.SemaphoreType.DMA,
            pltpu.SemaphoreType.DMA,
            pltpu.SemaphoreType.DMA,
        ],
        compiler_params=pltpu.CompilerParams(use_tc_tiling_on_sc=True, needs_layout_passes=False),
    )(tblt, tailp, input_ids)

    out3 = pl.kernel(
        _gather_body,
        mesh=mesh,
        out_type=jax.ShapeDtypeStruct((_BATCH, _D, _SEQ), jnp.float32),
        scratch_types=[
            pltpu.VMEM((8, _SEQ), jnp.int32),
            pltpu.VMEM((8, _SEQ), jnp.int32),
            pltpu.VMEM((_IW, 128), jnp.float32),
            pltpu.VMEM((_IW, 128), jnp.float32),
            pltpu.VMEM((_D, _IW), jnp.float32),
            pltpu.VMEM((_D, _IW), jnp.float32),
            pltpu.SemaphoreType.DMA,
            pltpu.SemaphoreType.DMA,
            pltpu.SemaphoreType.DMA,
            pltpu.SemaphoreType.DMA,
        ],
        compiler_params=pltpu.CompilerParams(use_tc_tiling_on_sc=True, needs_layout_passes=False),
    )(input_ids, idxh, fmt)

    token_embeddings = out3.transpose(0, 2, 1)
    return (input_ids, token_embeddings, attention_mask)


# speedup vs baseline: 2.3284x; 1.5619x over previous
import jax
import jax.numpy as jnp
from jax import lax
from jax.experimental import pallas as pl
from jax.experimental.pallas import tpu as pltpu
from jax.experimental.pallas import tpu_sc as plsc

_D = 64
_BATCH = 1024
_SEQ = 1024
_B = _BATCH * _SEQ
_NC = 2
_NS = 16
_NW = _NC * _NS
_BPW = _B // _NW
_ROWS_PW = _BPW // _SEQ
_IW = 128
_CPR = _SEQ // _IW
_G = 512
_KPG = _G // _IW
_GPB = _SEQ // _G
_NGRP = _BPW // _G
_NPAIR = _NGRP // 2


def _emb_body(ids_hbm, table_hbm, out_hbm, idx_v, rows0, rows1, gsem0, gsem1,
              wsem0, wsem1):
    wid = lax.axis_index("s") * _NC + lax.axis_index("c")
    b_base = wid * _ROWS_PW
    pltpu.sync_copy(ids_hbm.at[pl.ds(b_base, _ROWS_PW)], idx_v)

    def fire_g(g, buf, sem):
        for j in range(_KPG):
            k = g * _KPG + j
            pltpu.async_copy(
                table_hbm.at[idx_v.at[k // _CPR, pl.ds((k % _CPR) * _IW, _IW)]],
                buf.at[pl.ds(j * _IW, _IW)], sem)

    def drain_g(buf, sem):
        pltpu.make_async_copy(out_hbm.at[0, pl.ds(0, _G)], buf, sem).wait()

    def fire_w(g, buf, sem):
        b = b_base + g // _GPB
        s0 = (g % _GPB) * _G
        pltpu.async_copy(buf, out_hbm.at[b, pl.ds(s0, _G)], sem)

    def drain_w(buf, sem):
        pltpu.make_async_copy(buf, out_hbm.at[0, pl.ds(0, _G)], sem).wait()

    fire_g(0, rows0, gsem0)
    drain_g(rows0, gsem0)
    fire_g(1, rows1, gsem1)
    fire_w(0, rows0, wsem0)
    drain_g(rows1, gsem1)
    drain_w(rows0, wsem0)
    fire_g(2, rows0, gsem0)
    fire_w(1, rows1, wsem1)

    def pair(k, carry):
        g0 = 2 * k
        drain_g(rows0, gsem0)
        drain_w(rows1, wsem1)
        fire_g(g0 + 1, rows1, gsem1)
        fire_w(g0, rows0, wsem0)
        drain_g(rows1, gsem1)
        drain_w(rows0, wsem0)
        fire_g(g0 + 2, rows0, gsem0)
        fire_w(g0 + 1, rows1, wsem1)
        return carry

    lax.fori_loop(1, _NPAIR - 1, pair, 0)

    drain_g(rows0, gsem0)
    drain_w(rows1, wsem1)
    fire_g(_NGRP - 1, rows1, gsem1)
    fire_w(_NGRP - 2, rows0, wsem0)
    drain_g(rows1, gsem1)
    drain_w(rows0, wsem0)
    fire_w(_NGRP - 1, rows1, wsem1)
    drain_w(rows1, wsem1)


@jax.jit
def kernel(input_ids, attention_mask, embedding_weight):
    token_embeddings = pl.kernel(
        _emb_body,
        mesh=plsc.VectorSubcoreMesh(core_axis_name="c", subcore_axis_name="s"),
        out_type=jax.ShapeDtypeStruct((_BATCH, _SEQ, _D), jnp.float32),
        scratch_types=[
            pltpu.VMEM((_ROWS_PW, _SEQ), jnp.int32),
            pltpu.VMEM((_G, _D), jnp.float32),
            pltpu.VMEM((_G, _D), jnp.float32),
            pltpu.SemaphoreType.DMA,
            pltpu.SemaphoreType.DMA,
            pltpu.SemaphoreType.DMA,
            pltpu.SemaphoreType.DMA,
        ],
        compiler_params=pltpu.CompilerParams(use_tc_tiling_on_sc=False),
    )(input_ids, embedding_weight)
    return (input_ids, token_embeddings, attention_mask)
